# baseline (device time: 164763 ns/iter reference)
import jax
import jax.numpy as jnp
from jax import lax
from jax.experimental import pallas as pl
from jax.experimental.pallas import tpu as pltpu

N_DEV = 4
HQ = 32
HQ_LOC = 8
HALF = HQ_LOC // 2
SQ = 1024
SKV_EFF = 1024
DH = 128
DM = 1024
BLK = 64
SCALE = 0.08838834764831843


def kernel(x, Wq, K_ext, V_ext, Wo):
    bf16 = jnp.bfloat16
    f32 = jnp.float32
    i8 = jnp.int8
    Kf = K_ext[0].reshape(SKV_EFF, HQ * DH)
    Vf = V_ext[0].reshape(SKV_EFF, HQ * DH)

    def body(x_ref, wq_ref, kf_ref, vf_ref, wo_ref, out_ref,
             q_scr, stage, kq_full, ks_full, vq_full, vs_full,
             k_work, ks_work, v_work, vs_work,
             k_rel, ks_rel, v_rel, vs_rel,
             ctx_scr, comm, comm_l,
             stage_sems, kv_send_sems, kv_recv_sems, rel_recv_sems,
             fwd_send_sems, p2_recv_sems, ring_send_sems, ring_recv_sems,
             ring_l_send_sems, ring_l_recv_sems,
             ring_barrier_sem, exit_sem):
        my = lax.axis_index("i")
        left = lax.rem(my + N_DEV - 1, N_DEV)
        right = lax.rem(my + 1, N_DEV)

        def rc(src, dst, send_sem, recv_sem, j):
            return pltpu.make_async_remote_copy(
                src_ref=src, dst_ref=dst,
                send_sem=send_sem, recv_sem=recv_sem,
                device_id=(j,), device_id_type=pl.DeviceIdType.MESH,
            )

        def build_dev0():
            ds = pl.ds
            plan = [
                (kq_full.at[ds(2 * HQ_LOC, HALF)], k_rel, rel_recv_sems.at[0], 1),
                (vq_full.at[ds(2 * HQ_LOC, HALF)], v_rel, rel_recv_sems.at[1], 1),
                (ks_full.at[ds(2, 1)], ks_rel, rel_recv_sems.at[2], 1),
                (vs_full.at[ds(2, 1)], vs_rel, rel_recv_sems.at[3], 1),
                (kq_full.at[ds(2 * HQ_LOC + HALF, HALF)], k_rel,
                 rel_recv_sems.at[0], 3),
                (vq_full.at[ds(2 * HQ_LOC + HALF, HALF)], v_rel,
                 rel_recv_sems.at[1], 3),
                (kq_full.at[ds(HQ_LOC, HQ_LOC)], k_work, kv_recv_sems.at[0], 1),
                (vq_full.at[ds(HQ_LOC, HQ_LOC)], v_work, kv_recv_sems.at[1], 1),
                (ks_full.at[ds(1, 1)], ks_work, kv_recv_sems.at[2], 1),
                (vs_full.at[ds(1, 1)], vs_work, kv_recv_sems.at[3], 1),
                (kq_full.at[ds(3 * HQ_LOC, HQ_LOC)], k_work,
                 kv_recv_sems.at[0], 3),
                (vq_full.at[ds(3 * HQ_LOC, HQ_LOC)], v_work,
                 kv_recv_sems.at[1], 3),
                (ks_full.at[ds(3, 1)], ks_work, kv_recv_sems.at[2], 3),
                (vs_full.at[ds(3, 1)], vs_work, kv_recv_sems.at[3], 3),
            ]
            return [rc(s, d, kv_send_sems.at[i], r, j)
                    for i, (s, d, r, j) in enumerate(plan)]

        def fwd_k(off, slot):
            return rc(k_rel, k_work.at[pl.ds(off, HALF)],
                      fwd_send_sems.at[0], p2_recv_sems.at[slot], 2)

        def fwd_v(off, slot):
            return rc(v_rel, v_work.at[pl.ds(off, HALF)],
                      fwd_send_sems.at[1], p2_recv_sems.at[slot], 2)

        def fwd_ks():
            return rc(ks_rel, ks_work, fwd_send_sems.at[2],
                      p2_recv_sems.at[4], 2)

        def fwd_vs():
            return rc(vs_rel, vs_work, fwd_send_sems.at[3],
                      p2_recv_sems.at[5], 2)

        def rel_recv(t):
            srcs = [kq_full.at[pl.ds(0, HALF)], vq_full.at[pl.ds(0, HALF)],
                    ks_full.at[pl.ds(0, 1)], vs_full.at[pl.ds(0, 1)]]
            dsts = [k_rel, v_rel, ks_rel, vs_rel]
            return rc(srcs[t], dsts[t], kv_send_sems.at[t],
                      rel_recv_sems.at[t], 0)

        def own_recv(t):
            srcs = [kq_full.at[pl.ds(0, HQ_LOC)], vq_full.at[pl.ds(0, HQ_LOC)],
                    ks_full.at[pl.ds(0, 1)], vs_full.at[pl.ds(0, 1)]]
            dsts = [k_work, v_work, ks_work, vs_work]
            return rc(srcs[t], dsts[t], kv_send_sems.at[t],
                      kv_recv_sems.at[t], 0)

        barrier = pltpu.get_barrier_semaphore()
        for k in range(1, N_DEV):
            pl.semaphore_signal(
                barrier, inc=1,
                device_id=(lax.rem(my + k, N_DEV),),
                device_id_type=pl.DeviceIdType.MESH,
            )
        pl.semaphore_wait(barrier, N_DEV - 1)

        GRP = 8

        @pl.when(my == 0)
        def _():
            def chunk_copy(t, c, slot):
                src = kf_ref if t == 0 else vf_ref
                return pltpu.make_async_copy(
                    src.at[:, pl.ds(c * GRP * DH, GRP * DH)],
                    stage.at[slot],
                    stage_sems.at[slot],
                )

            chunks = [(t, c) for t in range(2) for c in range(HQ // GRP)]
            chunk_copy(*chunks[0], 0).start()
            for i, (t, c) in enumerate(chunks):
                slot = i % 2
                if i + 1 < len(chunks):
                    chunk_copy(*chunks[i + 1], (i + 1) % 2).start()
                chunk_copy(t, c, slot).wait()
                qdst = kq_full if t == 0 else vq_full
                sdst = ks_full if t == 0 else vs_full
                for hh in range(GRP):
                    h = c * GRP + hh
                    blk = stage[slot, :, hh * DH:(hh + 1) * DH]
                    mx = jnp.max(jnp.abs(blk)) + 1e-12
                    qdst[h] = jnp.round(blk * (127.0 / mx)).astype(i8)
                    sdst[h // HQ_LOC:h // HQ_LOC + 1,
                         h % HQ_LOC:h % HQ_LOC + 1] = (
                        (mx * (1.0 / 127.0)).reshape(1, 1))
            for r in build_dev0():
                r.start()

        @pl.when(my == 1)
        def _():
            rel_recv(0).wait_recv()
            fwd_k(0, 0).start()
            rel_recv(1).wait_recv()
            fwd_v(0, 2).start()
            rel_recv(2).wait_recv()
            fwd_ks().start()
            rel_recv(3).wait_recv()
            fwd_vs().start()

        @pl.when(my == 3)
        def _():
            rel_recv(0).wait_recv()
            fwd_k(HALF, 1).start()
            rel_recv(1).wait_recv()
            fwd_v(HALF, 3).start()

        q_scr[...] = jnp.dot(
            x_ref[...].astype(bf16), wq_ref[...].astype(bf16),
            preferred_element_type=f32,
        ).astype(bf16)

        @pl.when(my == 0)
        def _():
            k_work[...] = kq_full[0:HQ_LOC]
            ks_work[...] = ks_full[0:1]
            v_work[...] = vq_full[0:HQ_LOC]
            vs_work[...] = vs_full[0:1]

        @pl.when(jnp.logical_or(my == 1, my == 3))
        def _():
            for t in range(4):
                own_recv(t).wait_recv()

        @pl.when(my == 2)
        def _():
            fwd_k(0, 0).wait_recv()
            fwd_k(HALF, 1).wait_recv()
            fwd_v(0, 2).wait_recv()
            fwd_v(HALF, 3).wait_recv()
            fwd_ks().wait_recv()
            fwd_vs().wait_recv()

        rows = lax.broadcasted_iota(jnp.int32, (SQ, SKV_EFF), 0)
        cols = lax.broadcasted_iota(jnp.int32, (SQ, SKV_EFF), 1)
        mask = (cols // BLK) <= (rows // BLK)
        for h in range(HQ_LOC):
            q = q_scr[:, h * DH:(h + 1) * DH]
            kh = (k_work[h].astype(f32) * ks_work[0:1, h:h + 1]).astype(bf16)
            s = lax.dot_general(
                q, kh, (((1,), (1,)), ((), ())),
                preferred_element_type=f32,
            ) * SCALE
            s = jnp.where(mask, s, jnp.float32(-1e9))
            m = jnp.max(s, axis=1, keepdims=True)
            w = jnp.exp(s - m)
            dnm = jnp.sum(w, axis=1, keepdims=True)
            wn = (w * jnp.reciprocal(dnm)).astype(bf16)
            vh = (v_work[h].astype(f32) * vs_work[0:1, h:h + 1]).astype(bf16)
            ctx = jnp.dot(wn, vh, preferred_element_type=f32)
            ctx_scr[:, h * DH:(h + 1) * DH] = ctx.astype(bf16)

        out_ref[0, :, :] = jnp.dot(
            ctx_scr[...], wo_ref[...].astype(bf16),
            preferred_element_type=f32,
        )

        @pl.when(my == 0)
        def _():
            for r in build_dev0():
                r.wait_send()

        @pl.when(my == 1)
        def _():
            fwd_k(0, 0).wait_send()
            fwd_v(0, 2).wait_send()
            fwd_ks().wait_send()
            fwd_vs().wait_send()

        @pl.when(my == 3)
        def _():
            fwd_k(HALF, 1).wait_send()
            fwd_v(HALF, 3).wait_send()

        comm[0, :, :] = out_ref[0, 0:SQ // 2, :].astype(bf16)
        comm_l[0, :, :] = out_ref[0, SQ // 2:SQ, :].astype(bf16)

        for nbr in (left, right):
            pl.semaphore_signal(
                ring_barrier_sem, inc=1,
                device_id=(nbr,), device_id_type=pl.DeviceIdType.MESH,
            )
        pl.semaphore_wait(ring_barrier_sem, 2)

        for hop in range(N_DEV - 1):
            s_slot = hop % 2
            r_slot = (hop + 1) % 2
            rdma_r = pltpu.make_async_remote_copy(
                src_ref=comm.at[s_slot],
                dst_ref=comm.at[r_slot],
                send_sem=ring_send_sems.at[s_slot],
                recv_sem=ring_recv_sems.at[r_slot],
                device_id=(right,),
                device_id_type=pl.DeviceIdType.MESH,
            )
            rdma_l = pltpu.make_async_remote_copy(
                src_ref=comm_l.at[s_slot],
                dst_ref=comm_l.at[r_slot],
                send_sem=ring_l_send_sems.at[s_slot],
                recv_sem=ring_l_recv_sems.at[r_slot],
                device_id=(left,),
                device_id_type=pl.DeviceIdType.MESH,
            )
            rdma_r.start()
            rdma_l.start()
            rdma_r.wait()
            rdma_l.wait()
            out_ref[0, 0:SQ // 2, :] = (
                out_ref[0, 0:SQ // 2, :] + comm[r_slot].astype(f32))
            out_ref[0, SQ // 2:SQ, :] = (
                out_ref[0, SQ // 2:SQ, :] + comm_l[r_slot].astype(f32))

        for k in range(1, N_DEV):
            pl.semaphore_signal(
                exit_sem, inc=1,
                device_id=(lax.rem(my + k, N_DEV),),
                device_id_type=pl.DeviceIdType.MESH,
            )
        pl.semaphore_wait(exit_sem, N_DEV - 1)

    return pl.pallas_call(
        body,
        out_shape=jax.ShapeDtypeStruct((1, SQ, DM), jnp.float32),
        in_specs=[
            pl.BlockSpec(memory_space=pltpu.VMEM),
            pl.BlockSpec(memory_space=pltpu.VMEM),
            pl.BlockSpec(memory_space=pltpu.MemorySpace.HBM),
            pl.BlockSpec(memory_space=pltpu.MemorySpace.HBM),
            pl.BlockSpec(memory_space=pltpu.VMEM),
        ],
        out_specs=pl.BlockSpec(memory_space=pltpu.VMEM),
        scratch_shapes=[
            pltpu.VMEM((SQ, HQ_LOC * DH), bf16),
            pltpu.VMEM((2, SKV_EFF, 8 * DH), f32),
            pltpu.VMEM((HQ, SKV_EFF, DH), i8),
            pltpu.VMEM((N_DEV, DH), f32),
            pltpu.VMEM((HQ, SKV_EFF, DH), i8),
            pltpu.VMEM((N_DEV, DH), f32),
            pltpu.VMEM((HQ_LOC, SKV_EFF, DH), i8),
            pltpu.VMEM((1, DH), f32),
            pltpu.VMEM((HQ_LOC, SKV_EFF, DH), i8),
            pltpu.VMEM((1, DH), f32),
            pltpu.VMEM((HALF, SKV_EFF, DH), i8),
            pltpu.VMEM((1, DH), f32),
            pltpu.VMEM((HALF, SKV_EFF, DH), i8),
            pltpu.VMEM((1, DH), f32),
            pltpu.VMEM((SQ, HQ_LOC * DH), bf16),
            pltpu.VMEM((2, SQ // 2, DM), bf16),
            pltpu.VMEM((2, SQ // 2, DM), bf16),
            pltpu.SemaphoreType.DMA((2,)),
            pltpu.SemaphoreType.DMA((14,)),
            pltpu.SemaphoreType.DMA((4,)),
            pltpu.SemaphoreType.DMA((4,)),
            pltpu.SemaphoreType.DMA((4,)),
            pltpu.SemaphoreType.DMA((6,)),
            pltpu.SemaphoreType.DMA((2,)),
            pltpu.SemaphoreType.DMA((2,)),
            pltpu.SemaphoreType.DMA((2,)),
            pltpu.SemaphoreType.DMA((2,)),
            pltpu.SemaphoreType.REGULAR,
            pltpu.SemaphoreType.REGULAR,
        ],
        compiler_params=pltpu.CompilerParams(
            collective_id=0, vmem_limit_bytes=100 * 1024 * 1024),
    )(x[0], Wq, Kf, Vf, Wo)


# device time: 151055 ns/iter; 1.0907x vs baseline; 1.0907x over previous
import jax
import jax.numpy as jnp
from jax import lax
from jax.experimental import pallas as pl
from jax.experimental.pallas import tpu as pltpu

N_DEV = 4
HQ = 32
HQ_LOC = 8
HALF = HQ_LOC // 2
SQ = 1024
SKV_EFF = 1024
DH = 128
DM = 1024
BLK = 64
SCALE = 0.08838834764831843


def kernel(x, Wq, K_ext, V_ext, Wo):
    bf16 = jnp.bfloat16
    f32 = jnp.float32
    i8 = jnp.int8
    Kf = K_ext[0]
    Vf = V_ext[0]

    def body(x_ref, wq_ref, kf_ref, vf_ref, wo_ref, out_ref,
             q_scr, stage, kq_full, ks_full, vq_full, vs_full,
             k_work, ks_work, v_work, vs_work,
             k_rel, ks_rel, v_rel, vs_rel,
             ctx_scr, comm, comm_l,
             stage_sems, kv_send_sems, kv_recv_sems, rel_recv_sems,
             fwd_send_sems, p2_recv_sems, ring_send_sems, ring_recv_sems,
             ring_l_send_sems, ring_l_recv_sems,
             ring_barrier_sem, exit_sem):
        my = lax.axis_index("i")
        left = lax.rem(my + N_DEV - 1, N_DEV)
        right = lax.rem(my + 1, N_DEV)

        def rc(src, dst, send_sem, recv_sem, j):
            return pltpu.make_async_remote_copy(
                src_ref=src, dst_ref=dst,
                send_sem=send_sem, recv_sem=recv_sem,
                device_id=(j,), device_id_type=pl.DeviceIdType.MESH,
            )

        def build_dev0():
            ds = pl.ds
            plan = [
                (kq_full.at[ds(2 * HQ_LOC, HALF)], k_rel, rel_recv_sems.at[0], 1),
                (vq_full.at[ds(2 * HQ_LOC, HALF)], v_rel, rel_recv_sems.at[1], 1),
                (ks_full.at[ds(2, 1)], ks_rel, rel_recv_sems.at[2], 1),
                (vs_full.at[ds(2, 1)], vs_rel, rel_recv_sems.at[3], 1),
                (kq_full.at[ds(2 * HQ_LOC + HALF, HALF)], k_rel,
                 rel_recv_sems.at[0], 3),
                (vq_full.at[ds(2 * HQ_LOC + HALF, HALF)], v_rel,
                 rel_recv_sems.at[1], 3),
                (kq_full.at[ds(HQ_LOC, HQ_LOC)], k_work, kv_recv_sems.at[0], 1),
                (vq_full.at[ds(HQ_LOC, HQ_LOC)], v_work, kv_recv_sems.at[1], 1),
                (ks_full.at[ds(1, 1)], ks_work, kv_recv_sems.at[2], 1),
                (vs_full.at[ds(1, 1)], vs_work, kv_recv_sems.at[3], 1),
                (kq_full.at[ds(3 * HQ_LOC, HQ_LOC)], k_work,
                 kv_recv_sems.at[0], 3),
                (vq_full.at[ds(3 * HQ_LOC, HQ_LOC)], v_work,
                 kv_recv_sems.at[1], 3),
                (ks_full.at[ds(3, 1)], ks_work, kv_recv_sems.at[2], 3),
                (vs_full.at[ds(3, 1)], vs_work, kv_recv_sems.at[3], 3),
            ]
            return [rc(s, d, kv_send_sems.at[i], r, j)
                    for i, (s, d, r, j) in enumerate(plan)]

        def fwd_k(off, slot):
            return rc(k_rel, k_work.at[pl.ds(off, HALF)],
                      fwd_send_sems.at[0], p2_recv_sems.at[slot], 2)

        def fwd_v(off, slot):
            return rc(v_rel, v_work.at[pl.ds(off, HALF)],
                      fwd_send_sems.at[1], p2_recv_sems.at[slot], 2)

        def fwd_ks():
            return rc(ks_rel, ks_work, fwd_send_sems.at[2],
                      p2_recv_sems.at[4], 2)

        def fwd_vs():
            return rc(vs_rel, vs_work, fwd_send_sems.at[3],
                      p2_recv_sems.at[5], 2)

        def rel_recv(t):
            srcs = [kq_full.at[pl.ds(0, HALF)], vq_full.at[pl.ds(0, HALF)],
                    ks_full.at[pl.ds(0, 1)], vs_full.at[pl.ds(0, 1)]]
            dsts = [k_rel, v_rel, ks_rel, vs_rel]
            return rc(srcs[t], dsts[t], kv_send_sems.at[t],
                      rel_recv_sems.at[t], 0)

        def own_recv(t):
            srcs = [kq_full.at[pl.ds(0, HQ_LOC)], vq_full.at[pl.ds(0, HQ_LOC)],
                    ks_full.at[pl.ds(0, 1)], vs_full.at[pl.ds(0, 1)]]
            dsts = [k_work, v_work, ks_work, vs_work]
            return rc(srcs[t], dsts[t], kv_send_sems.at[t],
                      kv_recv_sems.at[t], 0)

        barrier = pltpu.get_barrier_semaphore()
        for k in range(1, N_DEV):
            pl.semaphore_signal(
                barrier, inc=1,
                device_id=(lax.rem(my + k, N_DEV),),
                device_id_type=pl.DeviceIdType.MESH,
            )
        pl.semaphore_wait(barrier, N_DEV - 1)

        @pl.when(my == 0)
        def _():
            chunks = [(t, h) for t in range(2) for h in range(HQ)]

            def chunk_copy(i, slot):
                t, h = chunks[i]
                src = kf_ref if t == 0 else vf_ref
                return pltpu.make_async_copy(
                    src.at[:, h, :], stage.at[slot], stage_sems.at[slot])

            chunk_copy(0, 0).start()
            for i, (t, h) in enumerate(chunks):
                slot = i % 2
                if i + 1 < len(chunks):
                    chunk_copy(i + 1, (i + 1) % 2).start()
                chunk_copy(i, slot).wait()
                qdst = kq_full if t == 0 else vq_full
                sdst = ks_full if t == 0 else vs_full
                blk = stage[slot]
                mx = jnp.max(jnp.abs(blk)) + 1e-12
                qdst[h] = jnp.round(blk * (127.0 / mx)).astype(i8)
                sdst[h // HQ_LOC:h // HQ_LOC + 1,
                     h % HQ_LOC:h % HQ_LOC + 1] = (
                    (mx * (1.0 / 127.0)).reshape(1, 1))
            for r in build_dev0():
                r.start()

        @pl.when(my == 1)
        def _():
            rel_recv(0).wait_recv()
            fwd_k(0, 0).start()
            rel_recv(1).wait_recv()
            fwd_v(0, 2).start()
            rel_recv(2).wait_recv()
            fwd_ks().start()
            rel_recv(3).wait_recv()
            fwd_vs().start()

        @pl.when(my == 3)
        def _():
            rel_recv(0).wait_recv()
            fwd_k(HALF, 1).start()
            rel_recv(1).wait_recv()
            fwd_v(HALF, 3).start()

        q_scr[...] = jnp.dot(
            x_ref[...].astype(bf16), wq_ref[...].astype(bf16),
            preferred_element_type=f32,
        ).astype(bf16)

        @pl.when(my == 0)
        def _():
            k_work[...] = kq_full[0:HQ_LOC]
            ks_work[...] = ks_full[0:1]
            v_work[...] = vq_full[0:HQ_LOC]
            vs_work[...] = vs_full[0:1]

        @pl.when(jnp.logical_or(my == 1, my == 3))
        def _():
            for t in range(4):
                own_recv(t).wait_recv()

        @pl.when(my == 2)
        def _():
            fwd_k(0, 0).wait_recv()
            fwd_k(HALF, 1).wait_recv()
            fwd_v(0, 2).wait_recv()
            fwd_v(HALF, 3).wait_recv()
            fwd_ks().wait_recv()
            fwd_vs().wait_recv()

        rows = lax.broadcasted_iota(jnp.int32, (SQ, SKV_EFF), 0)
        cols = lax.broadcasted_iota(jnp.int32, (SQ, SKV_EFF), 1)
        mask = (cols // BLK) <= (rows // BLK)
        for h in range(HQ_LOC):
            q = q_scr[:, h * DH:(h + 1) * DH]
            kh = (k_work[h].astype(f32) * ks_work[0:1, h:h + 1]).astype(bf16)
            s = lax.dot_general(
                q, kh, (((1,), (1,)), ((), ())),
                preferred_element_type=f32,
            ) * SCALE
            s = jnp.where(mask, s, jnp.float32(-1e9))
            m = jnp.max(s, axis=1, keepdims=True)
            w = jnp.exp(s - m)
            dnm = jnp.sum(w, axis=1, keepdims=True)
            wn = (w * jnp.reciprocal(dnm)).astype(bf16)
            vh = (v_work[h].astype(f32) * vs_work[0:1, h:h + 1]).astype(bf16)
            ctx = jnp.dot(wn, vh, preferred_element_type=f32)
            ctx_scr[:, h * DH:(h + 1) * DH] = ctx.astype(bf16)

        out_ref[0, :, :] = jnp.dot(
            ctx_scr[...], wo_ref[...].astype(bf16),
            preferred_element_type=f32,
        )

        @pl.when(my == 0)
        def _():
            for r in build_dev0():
                r.wait_send()

        @pl.when(my == 1)
        def _():
            fwd_k(0, 0).wait_send()
            fwd_v(0, 2).wait_send()
            fwd_ks().wait_send()
            fwd_vs().wait_send()

        @pl.when(my == 3)
        def _():
            fwd_k(HALF, 1).wait_send()
            fwd_v(HALF, 3).wait_send()

        comm[0, :, :] = out_ref[0, 0:SQ // 2, :].astype(bf16)
        comm_l[0, :, :] = out_ref[0, SQ // 2:SQ, :].astype(bf16)

        for nbr in (left, right):
            pl.semaphore_signal(
                ring_barrier_sem, inc=1,
                device_id=(nbr,), device_id_type=pl.DeviceIdType.MESH,
            )
        pl.semaphore_wait(ring_barrier_sem, 2)

        for hop in range(N_DEV - 1):
            s_slot = hop % 2
            r_slot = (hop + 1) % 2
            rdma_r = pltpu.make_async_remote_copy(
                src_ref=comm.at[s_slot],
                dst_ref=comm.at[r_slot],
                send_sem=ring_send_sems.at[s_slot],
                recv_sem=ring_recv_sems.at[r_slot],
                device_id=(right,),
                device_id_type=pl.DeviceIdType.MESH,
            )
            rdma_l = pltpu.make_async_remote_copy(
                src_ref=comm_l.at[s_slot],
                dst_ref=comm_l.at[r_slot],
                send_sem=ring_l_send_sems.at[s_slot],
                recv_sem=ring_l_recv_sems.at[r_slot],
                device_id=(left,),
                device_id_type=pl.DeviceIdType.MESH,
            )
            rdma_r.start()
            rdma_l.start()
            rdma_r.wait()
            rdma_l.wait()
            out_ref[0, 0:SQ // 2, :] = (
                out_ref[0, 0:SQ // 2, :] + comm[r_slot].astype(f32))
            out_ref[0, SQ // 2:SQ, :] = (
                out_ref[0, SQ // 2:SQ, :] + comm_l[r_slot].astype(f32))

        for k in range(1, N_DEV):
            pl.semaphore_signal(
                exit_sem, inc=1,
                device_id=(lax.rem(my + k, N_DEV),),
                device_id_type=pl.DeviceIdType.MESH,
            )
        pl.semaphore_wait(exit_sem, N_DEV - 1)

    return pl.pallas_call(
        body,
        out_shape=jax.ShapeDtypeStruct((1, SQ, DM), jnp.float32),
        in_specs=[
            pl.BlockSpec(memory_space=pltpu.VMEM),
            pl.BlockSpec(memory_space=pltpu.VMEM),
            pl.BlockSpec(memory_space=pltpu.MemorySpace.HBM),
            pl.BlockSpec(memory_space=pltpu.MemorySpace.HBM),
            pl.BlockSpec(memory_space=pltpu.VMEM),
        ],
        out_specs=pl.BlockSpec(memory_space=pltpu.VMEM),
        scratch_shapes=[
            pltpu.VMEM((SQ, HQ_LOC * DH), bf16),
            pltpu.VMEM((2, SKV_EFF, DH), f32),
            pltpu.VMEM((HQ, SKV_EFF, DH), i8),
            pltpu.VMEM((N_DEV, DH), f32),
            pltpu.VMEM((HQ, SKV_EFF, DH), i8),
            pltpu.VMEM((N_DEV, DH), f32),
            pltpu.VMEM((HQ_LOC, SKV_EFF, DH), i8),
            pltpu.VMEM((1, DH), f32),
            pltpu.VMEM((HQ_LOC, SKV_EFF, DH), i8),
            pltpu.VMEM((1, DH), f32),
            pltpu.VMEM((HALF, SKV_EFF, DH), i8),
            pltpu.VMEM((1, DH), f32),
            pltpu.VMEM((HALF, SKV_EFF, DH), i8),
            pltpu.VMEM((1, DH), f32),
            pltpu.VMEM((SQ, HQ_LOC * DH), bf16),
            pltpu.VMEM((2, SQ // 2, DM), bf16),
            pltpu.VMEM((2, SQ // 2, DM), bf16),
            pltpu.SemaphoreType.DMA((2,)),
            pltpu.SemaphoreType.DMA((14,)),
            pltpu.SemaphoreType.DMA((4,)),
            pltpu.SemaphoreType.DMA((4,)),
            pltpu.SemaphoreType.DMA((4,)),
            pltpu.SemaphoreType.DMA((6,)),
            pltpu.SemaphoreType.DMA((2,)),
            pltpu.SemaphoreType.DMA((2,)),
            pltpu.SemaphoreType.DMA((2,)),
            pltpu.SemaphoreType.DMA((2,)),
            pltpu.SemaphoreType.REGULAR,
            pltpu.SemaphoreType.REGULAR,
        ],
        compiler_params=pltpu.CompilerParams(
            collective_id=0, vmem_limit_bytes=100 * 1024 * 1024),
    )(x[0], Wq, Kf, Vf, Wo)


# device time: 119523 ns/iter; 1.3785x vs baseline; 1.2638x over previous
import jax
import jax.numpy as jnp
from jax import lax
from jax.experimental import pallas as pl
from jax.experimental.pallas import tpu as pltpu

N_DEV = 4
HQ = 32
HQ_LOC = 8
HALF = HQ_LOC // 2
SQ = 1024
SKV_EFF = 1024
DH = 128
DM = 1024
BLK = 64
SCALE = 0.08838834764831843


def kernel(x, Wq, K_ext, V_ext, Wo):
    bf16 = jnp.bfloat16
    f32 = jnp.float32
    i8 = jnp.int8
    Kf = K_ext[0]
    Vf = V_ext[0]

    def body(x_ref, wq_ref, kf_ref, vf_ref, wo_ref, out_ref,
             q_scr, stage, kq_full, ks_full, vq_full, vs_full,
             k_work, ks_work, v_work, vs_work,
             k_rel, ks_rel, v_rel, vs_rel,
             ctx_scr, comm, comm_l,
             stage_sems, kv_send_sems, kv_recv_sems, rel_recv_sems,
             fwd_send_sems, p2_recv_sems, ring_send_sems, ring_recv_sems,
             ring_l_send_sems, ring_l_recv_sems,
             ring_barrier_sem, exit_sem):
        my = lax.axis_index("i")
        left = lax.rem(my + N_DEV - 1, N_DEV)
        right = lax.rem(my + 1, N_DEV)

        def rc(src, dst, send_sem, recv_sem, j):
            return pltpu.make_async_remote_copy(
                src_ref=src, dst_ref=dst,
                send_sem=send_sem, recv_sem=recv_sem,
                device_id=(j,), device_id_type=pl.DeviceIdType.MESH,
            )

        def build_dev0():
            ds = pl.ds
            plan = [
                (kq_full.at[ds(2 * HQ_LOC, HALF)], k_rel, rel_recv_sems.at[0], 1),
                (vq_full.at[ds(2 * HQ_LOC, HALF)], v_rel, rel_recv_sems.at[1], 1),
                (ks_full.at[ds(2, 1)], ks_rel, rel_recv_sems.at[2], 1),
                (vs_full.at[ds(2, 1)], vs_rel, rel_recv_sems.at[3], 1),
                (kq_full.at[ds(2 * HQ_LOC + HALF, HALF)], k_rel,
                 rel_recv_sems.at[0], 3),
                (vq_full.at[ds(2 * HQ_LOC + HALF, HALF)], v_rel,
                 rel_recv_sems.at[1], 3),
                (kq_full.at[ds(HQ_LOC, HQ_LOC)], k_work, kv_recv_sems.at[0], 1),
                (vq_full.at[ds(HQ_LOC, HQ_LOC)], v_work, kv_recv_sems.at[1], 1),
                (ks_full.at[ds(1, 1)], ks_work, kv_recv_sems.at[2], 1),
                (vs_full.at[ds(1, 1)], vs_work, kv_recv_sems.at[3], 1),
                (kq_full.at[ds(3 * HQ_LOC, HQ_LOC)], k_work,
                 kv_recv_sems.at[0], 3),
                (vq_full.at[ds(3 * HQ_LOC, HQ_LOC)], v_work,
                 kv_recv_sems.at[1], 3),
                (ks_full.at[ds(3, 1)], ks_work, kv_recv_sems.at[2], 3),
                (vs_full.at[ds(3, 1)], vs_work, kv_recv_sems.at[3], 3),
            ]
            return [rc(s, d, kv_send_sems.at[i], r, j)
                    for i, (s, d, r, j) in enumerate(plan)]

        def fwd_k(off, slot):
            return rc(k_rel, k_work.at[pl.ds(off, HALF)],
                      fwd_send_sems.at[0], p2_recv_sems.at[slot], 2)

        def fwd_v(off, slot):
            return rc(v_rel, v_work.at[pl.ds(off, HALF)],
                      fwd_send_sems.at[1], p2_recv_sems.at[slot], 2)

        def fwd_ks():
            return rc(ks_rel, ks_work, fwd_send_sems.at[2],
                      p2_recv_sems.at[4], 2)

        def fwd_vs():
            return rc(vs_rel, vs_work, fwd_send_sems.at[3],
                      p2_recv_sems.at[5], 2)

        def rel_recv(t):
            srcs = [kq_full.at[pl.ds(0, HALF)], vq_full.at[pl.ds(0, HALF)],
                    ks_full.at[pl.ds(0, 1)], vs_full.at[pl.ds(0, 1)]]
            dsts = [k_rel, v_rel, ks_rel, vs_rel]
            return rc(srcs[t], dsts[t], kv_send_sems.at[t],
                      rel_recv_sems.at[t], 0)

        def own_recv(t):
            srcs = [kq_full.at[pl.ds(0, HQ_LOC)], vq_full.at[pl.ds(0, HQ_LOC)],
                    ks_full.at[pl.ds(0, 1)], vs_full.at[pl.ds(0, 1)]]
            dsts = [k_work, v_work, ks_work, vs_work]
            return rc(srcs[t], dsts[t], kv_send_sems.at[t],
                      kv_recv_sems.at[t], 0)

        barrier = pltpu.get_barrier_semaphore()
        for k in range(1, N_DEV):
            pl.semaphore_signal(
                barrier, inc=1,
                device_id=(lax.rem(my + k, N_DEV),),
                device_id_type=pl.DeviceIdType.MESH,
            )
        pl.semaphore_wait(barrier, N_DEV - 1)

        @pl.when(my == 0)
        def _():
            descs = build_dev0()
            SCHED = [
                (0, 2 * HQ_LOC, 3 * HQ_LOC, (0, 4, 2)),
                (0, HQ_LOC, 2 * HQ_LOC, (6, 8)),
                (0, 3 * HQ_LOC, 4 * HQ_LOC, (10, 12)),
                (1, 2 * HQ_LOC, 3 * HQ_LOC, (1, 5, 3)),
                (1, HQ_LOC, 2 * HQ_LOC, (7, 9)),
                (1, 3 * HQ_LOC, 4 * HQ_LOC, (11, 13)),
                (0, 0, HQ_LOC, ()),
                (1, 0, HQ_LOC, ()),
            ]
            flat = []
            actions = {}
            for t, lo, hi, sends in SCHED:
                flat.extend((t, h) for h in range(lo, hi))
                actions[len(flat) - 1] = (t, lo, sends)

            def chunk_copy(i, slot):
                t, h = flat[i]
                src = kf_ref if t == 0 else vf_ref
                return pltpu.make_async_copy(
                    src.at[:, h, :], stage.at[slot], stage_sems.at[slot])

            chunk_copy(0, 0).start()
            for i, (t, h) in enumerate(flat):
                slot = i % 2
                if i + 1 < len(flat):
                    chunk_copy(i + 1, (i + 1) % 2).start()
                chunk_copy(i, slot).wait()
                qdst = kq_full if t == 0 else vq_full
                sdst = ks_full if t == 0 else vs_full
                blk = stage[slot]
                mx = jnp.max(jnp.abs(blk)) + 1e-12
                qdst[h] = jnp.round(blk * (127.0 / mx)).astype(i8)
                sdst[h // HQ_LOC:h // HQ_LOC + 1,
                     h % HQ_LOC:h % HQ_LOC + 1] = (
                    (mx * (1.0 / 127.0)).reshape(1, 1))
                if i in actions:
                    tt, lo, sends = actions[i]
                    for sidx in sends:
                        descs[sidx].start()
                    if lo == 0 and tt == 0:
                        k_work[...] = kq_full[0:HQ_LOC]
                        ks_work[...] = ks_full[0:1]
                    if lo == 0 and tt == 1:
                        v_work[...] = vq_full[0:HQ_LOC]
                        vs_work[...] = vs_full[0:1]

        @pl.when(my == 1)
        def _():
            rel_recv(0).wait_recv()
            fwd_k(0, 0).start()
            rel_recv(1).wait_recv()
            fwd_v(0, 2).start()
            rel_recv(2).wait_recv()
            fwd_ks().start()
            rel_recv(3).wait_recv()
            fwd_vs().start()

        @pl.when(my == 3)
        def _():
            rel_recv(0).wait_recv()
            fwd_k(HALF, 1).start()
            rel_recv(1).wait_recv()
            fwd_v(HALF, 3).start()

        q_scr[...] = jnp.dot(
            x_ref[...].astype(bf16), wq_ref[...].astype(bf16),
            preferred_element_type=f32,
        ).astype(bf16)

        @pl.when(jnp.logical_or(my == 1, my == 3))
        def _():
            for t in range(4):
                own_recv(t).wait_recv()

        @pl.when(my == 2)
        def _():
            fwd_k(0, 0).wait_recv()
            fwd_k(HALF, 1).wait_recv()
            fwd_v(0, 2).wait_recv()
            fwd_v(HALF, 3).wait_recv()
            fwd_ks().wait_recv()
            fwd_vs().wait_recv()

        rows = lax.broadcasted_iota(jnp.int32, (SQ, SKV_EFF), 0)
        cols = lax.broadcasted_iota(jnp.int32, (SQ, SKV_EFF), 1)
        mask = (cols // BLK) <= (rows // BLK)
        for h in range(HQ_LOC):
            q = q_scr[:, h * DH:(h + 1) * DH]
            kh = (k_work[h].astype(f32) * ks_work[0:1, h:h + 1]).astype(bf16)
            s = lax.dot_general(
                q, kh, (((1,), (1,)), ((), ())),
                preferred_element_type=f32,
            ) * SCALE
            s = jnp.where(mask, s, jnp.float32(-1e9))
            m = jnp.max(s, axis=1, keepdims=True)
            w = jnp.exp(s - m)
            dnm = jnp.sum(w, axis=1, keepdims=True)
            wn = (w * jnp.reciprocal(dnm)).astype(bf16)
            vh = (v_work[h].astype(f32) * vs_work[0:1, h:h + 1]).astype(bf16)
            ctx = jnp.dot(wn, vh, preferred_element_type=f32)
            ctx_scr[:, h * DH:(h + 1) * DH] = ctx.astype(bf16)

        out_ref[0, :, :] = jnp.dot(
            ctx_scr[...], wo_ref[...].astype(bf16),
            preferred_element_type=f32,
        )

        @pl.when(my == 0)
        def _():
            for r in build_dev0():
                r.wait_send()

        @pl.when(my == 1)
        def _():
            fwd_k(0, 0).wait_send()
            fwd_v(0, 2).wait_send()
            fwd_ks().wait_send()
            fwd_vs().wait_send()

        @pl.when(my == 3)
        def _():
            fwd_k(HALF, 1).wait_send()
            fwd_v(HALF, 3).wait_send()

        comm[0, :, :] = out_ref[0, 0:SQ // 2, :].astype(bf16)
        comm_l[0, :, :] = out_ref[0, SQ // 2:SQ, :].astype(bf16)

        for nbr in (left, right):
            pl.semaphore_signal(
                ring_barrier_sem, inc=1,
                device_id=(nbr,), device_id_type=pl.DeviceIdType.MESH,
            )
        pl.semaphore_wait(ring_barrier_sem, 2)

        for hop in range(N_DEV - 1):
            s_slot = hop % 2
            r_slot = (hop + 1) % 2
            rdma_r = pltpu.make_async_remote_copy(
                src_ref=comm.at[s_slot],
                dst_ref=comm.at[r_slot],
                send_sem=ring_send_sems.at[s_slot],
                recv_sem=ring_recv_sems.at[r_slot],
                device_id=(right,),
                device_id_type=pl.DeviceIdType.MESH,
            )
            rdma_l = pltpu.make_async_remote_copy(
                src_ref=comm_l.at[s_slot],
                dst_ref=comm_l.at[r_slot],
                send_sem=ring_l_send_sems.at[s_slot],
                recv_sem=ring_l_recv_sems.at[r_slot],
                device_id=(left,),
                device_id_type=pl.DeviceIdType.MESH,
            )
            rdma_r.start()
            rdma_l.start()
            rdma_r.wait()
            rdma_l.wait()
            out_ref[0, 0:SQ // 2, :] = (
                out_ref[0, 0:SQ // 2, :] + comm[r_slot].astype(f32))
            out_ref[0, SQ // 2:SQ, :] = (
                out_ref[0, SQ // 2:SQ, :] + comm_l[r_slot].astype(f32))

        for k in range(1, N_DEV):
            pl.semaphore_signal(
                exit_sem, inc=1,
                device_id=(lax.rem(my + k, N_DEV),),
                device_id_type=pl.DeviceIdType.MESH,
            )
        pl.semaphore_wait(exit_sem, N_DEV - 1)

    return pl.pallas_call(
        body,
        out_shape=jax.ShapeDtypeStruct((1, SQ, DM), jnp.float32),
        in_specs=[
            pl.BlockSpec(memory_space=pltpu.VMEM),
            pl.BlockSpec(memory_space=pltpu.VMEM),
            pl.BlockSpec(memory_space=pltpu.MemorySpace.HBM),
            pl.BlockSpec(memory_space=pltpu.MemorySpace.HBM),
            pl.BlockSpec(memory_space=pltpu.VMEM),
        ],
        out_specs=pl.BlockSpec(memory_space=pltpu.VMEM),
        scratch_shapes=[
            pltpu.VMEM((SQ, HQ_LOC * DH), bf16),
            pltpu.VMEM((2, SKV_EFF, DH), f32),
            pltpu.VMEM((HQ, SKV_EFF, DH), i8),
            pltpu.VMEM((N_DEV, DH), f32),
            pltpu.VMEM((HQ, SKV_EFF, DH), i8),
            pltpu.VMEM((N_DEV, DH), f32),
            pltpu.VMEM((HQ_LOC, SKV_EFF, DH), i8),
            pltpu.VMEM((1, DH), f32),
            pltpu.VMEM((HQ_LOC, SKV_EFF, DH), i8),
            pltpu.VMEM((1, DH), f32),
            pltpu.VMEM((HALF, SKV_EFF, DH), i8),
            pltpu.VMEM((1, DH), f32),
            pltpu.VMEM((HALF, SKV_EFF, DH), i8),
            pltpu.VMEM((1, DH), f32),
            pltpu.VMEM((SQ, HQ_LOC * DH), bf16),
            pltpu.VMEM((2, SQ // 2, DM), bf16),
            pltpu.VMEM((2, SQ // 2, DM), bf16),
            pltpu.SemaphoreType.DMA((2,)),
            pltpu.SemaphoreType.DMA((14,)),
            pltpu.SemaphoreType.DMA((4,)),
            pltpu.SemaphoreType.DMA((4,)),
            pltpu.SemaphoreType.DMA((4,)),
            pltpu.SemaphoreType.DMA((6,)),
            pltpu.SemaphoreType.DMA((2,)),
            pltpu.SemaphoreType.DMA((2,)),
            pltpu.SemaphoreType.DMA((2,)),
            pltpu.SemaphoreType.DMA((2,)),
            pltpu.SemaphoreType.REGULAR,
            pltpu.SemaphoreType.REGULAR,
        ],
        compiler_params=pltpu.CompilerParams(
            collective_id=0, vmem_limit_bytes=100 * 1024 * 1024),
    )(x[0], Wq, Kf, Vf, Wo)


# device time: 112587 ns/iter; 1.4634x vs baseline; 1.0616x over previous
import jax
import jax.numpy as jnp
from jax import lax
from jax.experimental import pallas as pl
from jax.experimental.pallas import tpu as pltpu

N_DEV = 4
HQ = 32
HQ_LOC = 8
HALF = HQ_LOC // 2
SQ = 1024
SKV_EFF = 1024
DH = 128
DM = 1024
BLK = 64
SCALE = 0.08838834764831843


def kernel(x, Wq, K_ext, V_ext, Wo):
    bf16 = jnp.bfloat16
    f32 = jnp.float32
    i8 = jnp.int8
    Kf = K_ext[0]
    Vf = V_ext[0]

    def body(x_ref, wq_ref, kf_ref, vf_ref, wo_ref, out_ref,
             q_scr, stage, kq_full, ks_full, vq_full, vs_full,
             k_work, ks_work, v_work, vs_work,
             k_rel, ks_rel, v_rel, vs_rel,
             ctx_scr, comm, comm_l,
             stage_sems, kv_send_sems, kv_recv_sems, rel_recv_sems,
             fwd_send_sems, p2_recv_sems, ring_send_sems, ring_recv_sems,
             ring_l_send_sems, ring_l_recv_sems,
             ring_barrier_sem, exit_sem):
        my = lax.axis_index("i")
        left = lax.rem(my + N_DEV - 1, N_DEV)
        right = lax.rem(my + 1, N_DEV)

        def rc(src, dst, send_sem, recv_sem, j):
            return pltpu.make_async_remote_copy(
                src_ref=src, dst_ref=dst,
                send_sem=send_sem, recv_sem=recv_sem,
                device_id=(j,), device_id_type=pl.DeviceIdType.MESH,
            )

        def build_dev0():
            ds = pl.ds
            plan = [
                (kq_full.at[ds(2 * HQ_LOC, HALF)], k_rel, rel_recv_sems.at[0], 1),
                (vq_full.at[ds(2 * HQ_LOC, HALF)], v_rel, rel_recv_sems.at[1], 1),
                (ks_full.at[ds(2, 1)], ks_rel, rel_recv_sems.at[2], 1),
                (vs_full.at[ds(2, 1)], vs_rel, rel_recv_sems.at[3], 1),
                (kq_full.at[ds(2 * HQ_LOC + HALF, HALF)], k_rel,
                 rel_recv_sems.at[0], 3),
                (vq_full.at[ds(2 * HQ_LOC + HALF, HALF)], v_rel,
                 rel_recv_sems.at[1], 3),
                (kq_full.at[ds(HQ_LOC, HQ_LOC)], k_work, kv_recv_sems.at[0], 1),
                (vq_full.at[ds(HQ_LOC, HQ_LOC)], v_work, kv_recv_sems.at[1], 1),
                (ks_full.at[ds(1, 1)], ks_work, kv_recv_sems.at[2], 1),
                (vs_full.at[ds(1, 1)], vs_work, kv_recv_sems.at[3], 1),
                (kq_full.at[ds(3 * HQ_LOC, HQ_LOC)], k_work,
                 kv_recv_sems.at[0], 3),
                (vq_full.at[ds(3 * HQ_LOC, HQ_LOC)], v_work,
                 kv_recv_sems.at[1], 3),
                (ks_full.at[ds(3, 1)], ks_work, kv_recv_sems.at[2], 3),
                (vs_full.at[ds(3, 1)], vs_work, kv_recv_sems.at[3], 3),
            ]
            return [rc(s, d, kv_send_sems.at[i], r, j)
                    for i, (s, d, r, j) in enumerate(plan)]

        def fwd_k(off, slot):
            return rc(k_rel, k_work.at[pl.ds(off, HALF)],
                      fwd_send_sems.at[0], p2_recv_sems.at[slot], 2)

        def fwd_v(off, slot):
            return rc(v_rel, v_work.at[pl.ds(off, HALF)],
                      fwd_send_sems.at[1], p2_recv_sems.at[slot], 2)

        def fwd_ks():
            return rc(ks_rel, ks_work, fwd_send_sems.at[2],
                      p2_recv_sems.at[4], 2)

        def fwd_vs():
            return rc(vs_rel, vs_work, fwd_send_sems.at[3],
                      p2_recv_sems.at[5], 2)

        def rel_recv(t):
            srcs = [kq_full.at[pl.ds(0, HALF)], vq_full.at[pl.ds(0, HALF)],
                    ks_full.at[pl.ds(0, 1)], vs_full.at[pl.ds(0, 1)]]
            dsts = [k_rel, v_rel, ks_rel, vs_rel]
            return rc(srcs[t], dsts[t], kv_send_sems.at[t],
                      rel_recv_sems.at[t], 0)

        def own_recv(t):
            srcs = [kq_full.at[pl.ds(0, HQ_LOC)], vq_full.at[pl.ds(0, HQ_LOC)],
                    ks_full.at[pl.ds(0, 1)], vs_full.at[pl.ds(0, 1)]]
            dsts = [k_work, v_work, ks_work, vs_work]
            return rc(srcs[t], dsts[t], kv_send_sems.at[t],
                      kv_recv_sems.at[t], 0)

        barrier = pltpu.get_barrier_semaphore()
        for k in range(1, N_DEV):
            pl.semaphore_signal(
                barrier, inc=1,
                device_id=(lax.rem(my + k, N_DEV),),
                device_id_type=pl.DeviceIdType.MESH,
            )
        pl.semaphore_wait(barrier, N_DEV - 1)

        @pl.when(my == 0)
        def _():
            descs = build_dev0()
            SCHED = [
                (0, 2 * HQ_LOC, 3 * HQ_LOC, (0, 4, 2)),
                (0, HQ_LOC, 2 * HQ_LOC, (6, 8)),
                (0, 3 * HQ_LOC, 4 * HQ_LOC, (10, 12)),
                (1, 2 * HQ_LOC, 3 * HQ_LOC, (1, 5, 3)),
                (1, HQ_LOC, 2 * HQ_LOC, (7, 9)),
                (1, 3 * HQ_LOC, 4 * HQ_LOC, (11, 13)),
                (0, 0, HQ_LOC, ()),
                (1, 0, HQ_LOC, ()),
            ]
            flat = []
            actions = {}
            for t, lo, hi, sends in SCHED:
                flat.extend((t, h) for h in range(lo, hi))
                actions[len(flat) - 1] = (t, lo, sends)

            def chunk_copy(i, slot):
                t, h = flat[i]
                src = kf_ref if t == 0 else vf_ref
                return pltpu.make_async_copy(
                    src.at[:, h, :], stage.at[slot], stage_sems.at[slot])

            NSLOT = 4
            for p in range(NSLOT - 1):
                chunk_copy(p, p).start()
            for i, (t, h) in enumerate(flat):
                slot = i % NSLOT
                if i + NSLOT - 1 < len(flat):
                    chunk_copy(i + NSLOT - 1, (i + NSLOT - 1) % NSLOT).start()
                chunk_copy(i, slot).wait()
                qdst = kq_full if t == 0 else vq_full
                sdst = ks_full if t == 0 else vs_full
                blk = stage[slot]
                mx = jnp.max(jnp.abs(blk)) + 1e-12
                qdst[h] = jnp.round(blk * (127.0 / mx)).astype(i8)
                sdst[h // HQ_LOC:h // HQ_LOC + 1,
                     h % HQ_LOC:h % HQ_LOC + 1] = (
                    (mx * (1.0 / 127.0)).reshape(1, 1))
                if i in actions:
                    tt, lo, sends = actions[i]
                    for sidx in sends:
                        descs[sidx].start()
                    if lo == 0 and tt == 0:
                        k_work[...] = kq_full[0:HQ_LOC]
                        ks_work[...] = ks_full[0:1]
                    if lo == 0 and tt == 1:
                        v_work[...] = vq_full[0:HQ_LOC]
                        vs_work[...] = vs_full[0:1]

        @pl.when(my == 1)
        def _():
            rel_recv(0).wait_recv()
            fwd_k(0, 0).start()
            rel_recv(1).wait_recv()
            fwd_v(0, 2).start()
            rel_recv(2).wait_recv()
            fwd_ks().start()
            rel_recv(3).wait_recv()
            fwd_vs().start()

        @pl.when(my == 3)
        def _():
            rel_recv(0).wait_recv()
            fwd_k(HALF, 1).start()
            rel_recv(1).wait_recv()
            fwd_v(HALF, 3).start()

        q_scr[...] = jnp.dot(
            x_ref[...].astype(bf16), wq_ref[...].astype(bf16),
            preferred_element_type=f32,
        ).astype(bf16)

        @pl.when(jnp.logical_or(my == 1, my == 3))
        def _():
            for t in range(4):
                own_recv(t).wait_recv()

        @pl.when(my == 2)
        def _():
            fwd_k(0, 0).wait_recv()
            fwd_k(HALF, 1).wait_recv()
            fwd_v(0, 2).wait_recv()
            fwd_v(HALF, 3).wait_recv()
            fwd_ks().wait_recv()
            fwd_vs().wait_recv()

        rows = lax.broadcasted_iota(jnp.int32, (SQ, SKV_EFF), 0)
        cols = lax.broadcasted_iota(jnp.int32, (SQ, SKV_EFF), 1)
        mask = (cols // BLK) <= (rows // BLK)
        for h in range(HQ_LOC):
            q = q_scr[:, h * DH:(h + 1) * DH]
            kh = (k_work[h].astype(f32) * ks_work[0:1, h:h + 1]).astype(bf16)
            s = lax.dot_general(
                q, kh, (((1,), (1,)), ((), ())),
                preferred_element_type=f32,
            ) * SCALE
            s = jnp.where(mask, s, jnp.float32(-1e9))
            m = jnp.max(s, axis=1, keepdims=True)
            w = jnp.exp(s - m)
            dnm = jnp.sum(w, axis=1, keepdims=True)
            wn = (w * jnp.reciprocal(dnm)).astype(bf16)
            vh = (v_work[h].astype(f32) * vs_work[0:1, h:h + 1]).astype(bf16)
            ctx = jnp.dot(wn, vh, preferred_element_type=f32)
            ctx_scr[:, h * DH:(h + 1) * DH] = ctx.astype(bf16)

        out_ref[0, :, :] = jnp.dot(
            ctx_scr[...], wo_ref[...].astype(bf16),
            preferred_element_type=f32,
        )

        @pl.when(my == 0)
        def _():
            for r in build_dev0():
                r.wait_send()

        @pl.when(my == 1)
        def _():
            fwd_k(0, 0).wait_send()
            fwd_v(0, 2).wait_send()
            fwd_ks().wait_send()
            fwd_vs().wait_send()

        @pl.when(my == 3)
        def _():
            fwd_k(HALF, 1).wait_send()
            fwd_v(HALF, 3).wait_send()

        comm[0, :, :] = out_ref[0, 0:SQ // 2, :].astype(bf16)
        comm_l[0, :, :] = out_ref[0, SQ // 2:SQ, :].astype(bf16)

        for nbr in (left, right):
            pl.semaphore_signal(
                ring_barrier_sem, inc=1,
                device_id=(nbr,), device_id_type=pl.DeviceIdType.MESH,
            )
        pl.semaphore_wait(ring_barrier_sem, 2)

        for hop in range(N_DEV - 1):
            s_slot = hop % 2
            r_slot = (hop + 1) % 2
            rdma_r = pltpu.make_async_remote_copy(
                src_ref=comm.at[s_slot],
                dst_ref=comm.at[r_slot],
                send_sem=ring_send_sems.at[s_slot],
                recv_sem=ring_recv_sems.at[r_slot],
                device_id=(right,),
                device_id_type=pl.DeviceIdType.MESH,
            )
            rdma_l = pltpu.make_async_remote_copy(
                src_ref=comm_l.at[s_slot],
                dst_ref=comm_l.at[r_slot],
                send_sem=ring_l_send_sems.at[s_slot],
                recv_sem=ring_l_recv_sems.at[r_slot],
                device_id=(left,),
                device_id_type=pl.DeviceIdType.MESH,
            )
            rdma_r.start()
            rdma_l.start()
            rdma_r.wait()
            rdma_l.wait()
            out_ref[0, 0:SQ // 2, :] = (
                out_ref[0, 0:SQ // 2, :] + comm[r_slot].astype(f32))
            out_ref[0, SQ // 2:SQ, :] = (
                out_ref[0, SQ // 2:SQ, :] + comm_l[r_slot].astype(f32))

        for k in range(1, N_DEV):
            pl.semaphore_signal(
                exit_sem, inc=1,
                device_id=(lax.rem(my + k, N_DEV),),
                device_id_type=pl.DeviceIdType.MESH,
            )
        pl.semaphore_wait(exit_sem, N_DEV - 1)

    return pl.pallas_call(
        body,
        out_shape=jax.ShapeDtypeStruct((1, SQ, DM), jnp.float32),
        in_specs=[
            pl.BlockSpec(memory_space=pltpu.VMEM),
            pl.BlockSpec(memory_space=pltpu.VMEM),
            pl.BlockSpec(memory_space=pltpu.MemorySpace.HBM),
            pl.BlockSpec(memory_space=pltpu.MemorySpace.HBM),
            pl.BlockSpec(memory_space=pltpu.VMEM),
        ],
        out_specs=pl.BlockSpec(memory_space=pltpu.VMEM),
        scratch_shapes=[
            pltpu.VMEM((SQ, HQ_LOC * DH), bf16),
            pltpu.VMEM((4, SKV_EFF, DH), f32),
            pltpu.VMEM((HQ, SKV_EFF, DH), i8),
            pltpu.VMEM((N_DEV, DH), f32),
            pltpu.VMEM((HQ, SKV_EFF, DH), i8),
            pltpu.VMEM((N_DEV, DH), f32),
            pltpu.VMEM((HQ_LOC, SKV_EFF, DH), i8),
            pltpu.VMEM((1, DH), f32),
            pltpu.VMEM((HQ_LOC, SKV_EFF, DH), i8),
            pltpu.VMEM((1, DH), f32),
            pltpu.VMEM((HALF, SKV_EFF, DH), i8),
            pltpu.VMEM((1, DH), f32),
            pltpu.VMEM((HALF, SKV_EFF, DH), i8),
            pltpu.VMEM((1, DH), f32),
            pltpu.VMEM((SQ, HQ_LOC * DH), bf16),
            pltpu.VMEM((2, SQ // 2, DM), bf16),
            pltpu.VMEM((2, SQ // 2, DM), bf16),
            pltpu.SemaphoreType.DMA((4,)),
            pltpu.SemaphoreType.DMA((14,)),
            pltpu.SemaphoreType.DMA((4,)),
            pltpu.SemaphoreType.DMA((4,)),
            pltpu.SemaphoreType.DMA((4,)),
            pltpu.SemaphoreType.DMA((6,)),
            pltpu.SemaphoreType.DMA((2,)),
            pltpu.SemaphoreType.DMA((2,)),
            pltpu.SemaphoreType.DMA((2,)),
            pltpu.SemaphoreType.DMA((2,)),
            pltpu.SemaphoreType.REGULAR,
            pltpu.SemaphoreType.REGULAR,
        ],
        compiler_params=pltpu.CompilerParams(
            collective_id=0, vmem_limit_bytes=100 * 1024 * 1024),
    )(x[0], Wq, Kf, Vf, Wo)
